# baseline (speedup 1.0000x reference)
import functools

import jax
import jax.numpy as jnp
from jax import lax
from jax.experimental import pallas as pl
from jax.experimental.pallas import tpu as pltpu
from jax.experimental.pallas import tpu_sc as plsc

LANES = 16


@functools.lru_cache(maxsize=None)
def _make_emb_kernel(B, S, D, V):
    N = B * S
    info = plsc.get_sparse_core_info()
    NC, NS = info.num_cores, info.num_subcores
    NW = NC * NS
    assert N % (NW * S) == 0, (N, NW, S)
    chunks_per_w = N // (NW * S)
    mesh = plsc.VectorSubcoreMesh(core_axis_name="c", subcore_axis_name="s")

    @functools.partial(
        pl.kernel,
        mesh=mesh,
        compiler_params=pltpu.CompilerParams(use_tc_tiling_on_sc=False),
        out_type=jax.ShapeDtypeStruct((N, D), jnp.float32),
        scratch_types=[
            pltpu.VMEM((S,), jnp.int32),
            pltpu.VMEM((S, D), jnp.float32),
            pltpu.VMEM((S, D), jnp.float32),
            pltpu.SemaphoreType.DMA,
        ],
    )
    def emb_kernel(idx_hbm, wte_hbm, wpe_hbm, out_hbm, idx_v, rows_v, wpe_v, sem):
        wid = lax.axis_index("s") * NC + lax.axis_index("c")
        pltpu.sync_copy(wpe_hbm, wpe_v)
        base = wid * chunks_per_w * S

        def chunk_body(c, _):
            off = base + c * S
            pltpu.sync_copy(idx_hbm.at[pl.ds(off, S)], idx_v)
            pltpu.async_copy(wte_hbm.at[idx_v], rows_v, sem).wait()

            def row_body(r, _):
                for j in range(D // LANES):
                    plsc.addupdate(
                        rows_v.at[r, pl.ds(j * LANES, LANES)],
                        wpe_v[r, pl.ds(j * LANES, LANES)],
                    )
                return 0

            lax.fori_loop(0, S, row_body, 0)
            pltpu.sync_copy(rows_v, out_hbm.at[pl.ds(off, S)])
            return 0

        lax.fori_loop(0, chunks_per_w, chunk_body, 0)

    return emb_kernel


def kernel(idx, wte, wpe):
    B, S = idx.shape
    V, D = wte.shape
    emb = _make_emb_kernel(B, S, D, V)(
        idx.reshape(-1).astype(jnp.int32), wte, wpe
    )
    return emb.reshape(B, S, D)


# speedup vs baseline: 1.0720x; 1.0720x over previous
import functools

import jax
import jax.numpy as jnp
from jax import lax
from jax.experimental import pallas as pl
from jax.experimental.pallas import tpu as pltpu
from jax.experimental.pallas import tpu_sc as plsc

LANES = 16
NBUF = 4
AHEAD = 2


@functools.lru_cache(maxsize=None)
def _make_emb_kernel(B, S, D, V):
    N = B * S
    info = plsc.get_sparse_core_info()
    NC, NS = info.num_cores, info.num_subcores
    NW = NC * NS
    assert N % (NW * S) == 0, (N, NW, S)
    nchunks = N // (NW * S)
    rows_per_w = nchunks * S
    mesh = plsc.VectorSubcoreMesh(core_axis_name="c", subcore_axis_name="s")

    @functools.partial(
        pl.kernel,
        mesh=mesh,
        compiler_params=pltpu.CompilerParams(use_tc_tiling_on_sc=False),
        out_type=jax.ShapeDtypeStruct((N, D), jnp.float32),
        scratch_types=[
            pltpu.VMEM((rows_per_w,), jnp.int32),
            pltpu.VMEM((S, D), jnp.float32),
            [pltpu.VMEM((S, D), jnp.float32)] * NBUF,
            [pltpu.SemaphoreType.DMA] * NBUF,
            [pltpu.SemaphoreType.DMA] * NBUF,
        ],
    )
    def emb_kernel(idx_hbm, wte_hbm, wpe_hbm, out_hbm,
                   idx_v, wpe_v, rows_v, g_sems, o_sems):
        wid = lax.axis_index("s") * NC + lax.axis_index("c")
        base = wid * rows_per_w
        pltpu.sync_copy(wpe_hbm, wpe_v)
        pltpu.sync_copy(idx_hbm.at[pl.ds(base, rows_per_w)], idx_v)

        def start_gather(c):
            b = c % NBUF
            return pltpu.async_copy(
                wte_hbm.at[idx_v.at[pl.ds(c * S, S)]], rows_v[b], g_sems[b])

        g_handles = {}
        o_handles = {}
        for c in range(min(AHEAD, nchunks)):
            g_handles[c] = start_gather(c)

        for c in range(nchunks):
            b = c % NBUF
            ca = c + AHEAD
            if ca < nchunks:
                if ca >= NBUF:
                    o_handles[ca - NBUF].wait()
                g_handles[ca] = start_gather(ca)
            g_handles[c].wait()

            @plsc.parallel_loop(0, S, 2, unroll=4)
            def _(r):
                for k in range(2):
                    for j in range(D // LANES):
                        plsc.addupdate(
                            rows_v[b].at[r + k, pl.ds(j * LANES, LANES)],
                            wpe_v[r + k, pl.ds(j * LANES, LANES)],
                        )

            o_handles[c] = pltpu.async_copy(
                rows_v[b], out_hbm.at[pl.ds(base + c * S, S)], o_sems[b])

        for c in range(max(0, nchunks - NBUF), nchunks):
            o_handles[c].wait()

    return emb_kernel


def kernel(idx, wte, wpe):
    B, S = idx.shape
    V, D = wte.shape
    emb = _make_emb_kernel(B, S, D, V)(
        idx.reshape(-1).astype(jnp.int32), wte, wpe
    )
    return emb.reshape(B, S, D)


# speedup vs baseline: 1.0737x; 1.0016x over previous
import functools

import jax
import jax.numpy as jnp
from jax import lax
from jax.experimental import pallas as pl
from jax.experimental.pallas import tpu as pltpu
from jax.experimental.pallas import tpu_sc as plsc

LANES = 16
NBUF = 4
AHEAD = 2


@functools.lru_cache(maxsize=None)
def _make_emb_kernel(B, S, D, V):
    N = B * S
    info = plsc.get_sparse_core_info()
    NC, NS = info.num_cores, info.num_subcores
    NW = NC * NS
    assert B % NW == 0, (B, NW)
    nchunks = B // NW
    mesh = plsc.VectorSubcoreMesh(core_axis_name="c", subcore_axis_name="s")

    @functools.partial(
        pl.kernel,
        mesh=mesh,
        compiler_params=pltpu.CompilerParams(use_tc_tiling_on_sc=False),
        out_type=jax.ShapeDtypeStruct((N, D), jnp.float32),
        scratch_types=[
            pltpu.VMEM((nchunks, S), jnp.int32),
            pltpu.VMEM((S, D), jnp.float32),
            [pltpu.VMEM((S, D), jnp.float32)] * NBUF,
            [pltpu.SemaphoreType.DMA] * NBUF,
            [pltpu.SemaphoreType.DMA] * NBUF,
        ],
    )
    def emb_kernel(idx_hbm, wte_hbm, wpe_hbm, out_hbm,
                   idx_v, wpe_v, rows_v, g_sems, o_sems):
        wid = lax.axis_index("s") * NC + lax.axis_index("c")
        row0 = wid * nchunks
        pltpu.sync_copy(wpe_hbm, wpe_v)
        pltpu.sync_copy(idx_hbm.at[pl.ds(row0, nchunks)], idx_v)

        def start_gather(c):
            b = c % NBUF
            return pltpu.async_copy(
                wte_hbm.at[idx_v.at[c]], rows_v[b], g_sems[b])

        g_handles = {}
        o_handles = {}
        for c in range(min(AHEAD, nchunks)):
            g_handles[c] = start_gather(c)

        for c in range(nchunks):
            b = c % NBUF
            ca = c + AHEAD
            if ca < nchunks:
                if ca >= NBUF:
                    o_handles[ca - NBUF].wait()
                g_handles[ca] = start_gather(ca)
            g_handles[c].wait()

            @plsc.parallel_loop(0, S, 2, unroll=4)
            def _(r):
                for k in range(2):
                    for j in range(D // LANES):
                        plsc.addupdate(
                            rows_v[b].at[r + k, pl.ds(j * LANES, LANES)],
                            wpe_v[r + k, pl.ds(j * LANES, LANES)],
                        )

            o_handles[c] = pltpu.async_copy(
                rows_v[b], out_hbm.at[pl.ds((row0 + c) * S, S)], o_sems[b])

        for c in range(max(0, nchunks - NBUF), nchunks):
            o_handles[c].wait()

    return emb_kernel


def kernel(idx, wte, wpe):
    B, S = idx.shape
    V, D = wte.shape
    emb = _make_emb_kernel(B, S, D, V)(idx, wte, wpe)
    return emb.reshape(B, S, D)
